# Initial kernel scaffold; baseline (speedup 1.0000x reference)
import jax
import jax.numpy as jnp
from jax.experimental import pallas as pl


def kernel(a, W_e, b_e, D, b_d):
    raise NotImplementedError("write your pallas kernel here")


# speedup vs baseline: 2.8800x; 2.8800x over previous
import functools

import jax
import jax.numpy as jnp
from jax.experimental import pallas as pl
from jax.experimental.pallas import tpu as pltpu

K_TOP = 64


def _encode_body(a_ref, w_ref, be_ref, s_ref):
    acc = jax.lax.dot_general(
        a_ref[...].astype(jnp.bfloat16), w_ref[...].astype(jnp.bfloat16),
        (((1,), (1,)), ((), ())),
        preferred_element_type=jnp.float32,
    )
    s_ref[...] = jnp.maximum(acc + be_ref[...], 0.0)


def _encode(a, w_e, b_e, br, bm):
    n, c = a.shape
    m = w_e.shape[0]
    grid = (n // br, m // bm)
    return pl.pallas_call(
        _encode_body,
        grid=grid,
        in_specs=[
            pl.BlockSpec((br, c), lambda i, j: (i, 0)),
            pl.BlockSpec((bm, c), lambda i, j: (j, 0)),
            pl.BlockSpec((1, bm), lambda i, j: (0, j)),
        ],
        out_specs=pl.BlockSpec((br, bm), lambda i, j: (i, j)),
        out_shape=jax.ShapeDtypeStruct((n, m), jnp.float32),
        compiler_params=pltpu.CompilerParams(
            dimension_semantics=("parallel", "parallel"),
        ),
    )(a, w_e, b_e.reshape(1, m))


def _topk_body(m, s_ref, vals_ref, idx_ref, sb, iot):
    r = sb.shape[0]
    sb[...] = s_ref[...]
    iot[...] = jax.lax.broadcasted_iota(jnp.int32, (r, m), 1)
    lane64 = jax.lax.broadcasted_iota(jnp.int32, (r, K_TOP), 1)
    vals_ref[...] = jnp.zeros((r, K_TOP), jnp.float32)
    idx_ref[...] = jnp.zeros((r, K_TOP), jnp.int32)

    def body(it, _):
        cur = sb[...]
        mx = jnp.max(cur, axis=1, keepdims=True)
        cand = jnp.where(cur == mx, iot[...], jnp.int32(2**30))
        am = jnp.min(cand, axis=1, keepdims=True)
        sel = lane64 == it
        vals_ref[...] = jnp.where(sel, mx, vals_ref[...])
        idx_ref[...] = jnp.where(sel, am, idx_ref[...])
        sb[...] = jnp.where(iot[...] == am, jnp.float32(-1.0), cur)
        return 0

    jax.lax.fori_loop(0, K_TOP, body, 0)


def _topk(s, br):
    n, m = s.shape
    return pl.pallas_call(
        functools.partial(_topk_body, m),
        grid=(n // br,),
        in_specs=[pl.BlockSpec((br, m), lambda i: (i, 0))],
        out_specs=[
            pl.BlockSpec((br, K_TOP), lambda i: (i, 0)),
            pl.BlockSpec((br, K_TOP), lambda i: (i, 0)),
        ],
        out_shape=[
            jax.ShapeDtypeStruct((n, K_TOP), jnp.float32),
            jax.ShapeDtypeStruct((n, K_TOP), jnp.int32),
        ],
        scratch_shapes=[
            pltpu.VMEM((br, m), jnp.float32),
            pltpu.VMEM((br, m), jnp.int32),
        ],
        compiler_params=pltpu.CompilerParams(
            dimension_semantics=("parallel",),
        ),
    )(s)


def _decode_body(s_ref, d_ref, t_ref, bd_ref, out_ref):
    kb = pl.program_id(1)
    cur = s_ref[...]
    z = (cur * (cur >= t_ref[...])).astype(jnp.bfloat16)
    part = jax.lax.dot_general(
        z, d_ref[...].astype(jnp.bfloat16),
        (((1,), (1,)), ((), ())),
        preferred_element_type=jnp.float32,
    )

    @pl.when(kb == 0)
    def _():
        out_ref[...] = part + bd_ref[...]

    @pl.when(kb > 0)
    def _():
        out_ref[...] += part


def _decode(s, d, t64, b_d, br, bk):
    n, m = s.shape
    c = d.shape[0]
    grid = (n // br, m // bk)
    return pl.pallas_call(
        _decode_body,
        grid=grid,
        in_specs=[
            pl.BlockSpec((br, bk), lambda i, j: (i, j)),
            pl.BlockSpec((c, bk), lambda i, j: (0, j)),
            pl.BlockSpec((br, 1), lambda i, j: (i, 0)),
            pl.BlockSpec((1, c), lambda i, j: (0, 0)),
        ],
        out_specs=pl.BlockSpec((br, c), lambda i, j: (i, 0)),
        out_shape=jax.ShapeDtypeStruct((n, c), jnp.float32),
        compiler_params=pltpu.CompilerParams(
            dimension_semantics=("parallel", "arbitrary"),
        ),
    )(s, d, t64, b_d.reshape(1, c))


def kernel(a, W_e, b_e, D, b_d):
    n, c = a.shape
    m = W_e.shape[0]

    br_enc = min(1024, n)
    bm_enc = min(512, m)
    s = _encode(a, W_e, b_e, br_enc, bm_enc)

    br_top = min(64, n)
    vals, idx = _topk(s, br_top)

    t64 = vals[:, K_TOP - 1:K_TOP]
    br_dec = min(512, n)
    bk_dec = min(1024, m)
    recon = _decode(s, D, t64, b_d, br_dec, bk_dec)
    return (recon, vals, idx)
